# baseline (speedup 1.0000x reference)
import jax
import jax.numpy as jnp
from jax.experimental import pallas as pl
from jax.experimental.pallas import tpu as pltpu

K_NN = 3
M = 3136
M_PAD = 3200
LANES = 128
N_CHUNK = M_PAD // LANES
D_PAD = 64
R_BLOCK = 896


def _knn_body(f_ref, mb_ref, out_ref, mba_ref):
    r = f_ref.shape[0]
    d = mb_ref.shape[0]

    @pl.when(pl.program_id(0) == 0)
    def _build_augmented_bank():
        mb = mb_ref[...]
        c_sq = jnp.sum(mb * mb, axis=0, keepdims=True)
        zeros = jnp.zeros((D_PAD - d - 1, M_PAD), jnp.float32)
        mba_ref[...] = jnp.concatenate([-2.0 * mb, c_sq, zeros], axis=0)

    f = f_ref[...]
    f_aug = jnp.concatenate(
        [f, jnp.ones((r, 1), jnp.float32),
         jnp.zeros((r, D_PAD - d - 1), jnp.float32)], axis=1)
    dot = jax.lax.dot_general(
        f_aug, mba_ref[...], (((1,), (0,)), ((), ())),
        preferred_element_type=jnp.float32)

    inf = jnp.float32(jnp.inf)
    lane = jax.lax.broadcasted_iota(jnp.int32, (r, LANES), 1)
    m1 = jnp.full((r, LANES), inf)
    m2 = jnp.full((r, LANES), inf)
    m3 = jnp.full((r, LANES), inf)
    for j in range(N_CHUNK):
        v = dot[:, j * LANES:(j + 1) * LANES]
        if (j + 1) * LANES > M:
            v = jnp.where(lane < M - j * LANES, v, inf)
        t = jnp.minimum(m1, v)
        v = jnp.maximum(m1, v)
        m1 = t
        t = jnp.minimum(m2, v)
        v = jnp.maximum(m2, v)
        m2 = t
        m3 = jnp.minimum(m3, v)

    w = LANES
    while w > 1:
        h = w // 2
        a1, b1 = m1[:, :h], m1[:, h:w]
        a2, b2 = m2[:, :h], m2[:, h:w]
        a3, b3 = m3[:, :h], m3[:, h:w]
        m1 = jnp.minimum(a1, b1)
        m2 = jnp.minimum(jnp.minimum(a2, b2), jnp.maximum(a1, b1))
        m3 = jnp.minimum(jnp.minimum(a3, b3),
                         jnp.minimum(jnp.maximum(a2, b1), jnp.maximum(a1, b2)))
        w = h

    f_sq = jnp.sum(f * f, axis=1, keepdims=True)
    out_ref[:, 0:1] = m1 + f_sq
    out_ref[:, 1:2] = m2 + f_sq
    out_ref[:, 2:3] = m3 + f_sq


@jax.jit
def kernel(target_oriented_features, memory_bank):
    B, HW, D = target_oriented_features.shape
    rows = B * HW
    f = target_oriented_features.reshape(rows, D)
    mb = jnp.pad(memory_bank, ((0, 0), (0, M_PAD - M)))
    out = pl.pallas_call(
        _knn_body,
        grid=(rows // R_BLOCK,),
        in_specs=[
            pl.BlockSpec((R_BLOCK, D), lambda i: (i, 0)),
            pl.BlockSpec((D, M_PAD), lambda i: (0, 0)),
        ],
        out_specs=pl.BlockSpec((R_BLOCK, K_NN), lambda i: (i, 0)),
        out_shape=jax.ShapeDtypeStruct((rows, K_NN), jnp.float32),
        scratch_shapes=[pltpu.VMEM((D_PAD, M_PAD), jnp.float32)],
    )(f, mb)
    return out.reshape(B, HW, K_NN)


# speedup vs baseline: 1.4218x; 1.4218x over previous
import jax
import jax.numpy as jnp
from jax.experimental import pallas as pl
from jax.experimental.pallas import tpu as pltpu

K_NN = 3
M = 3136
M_PAD = 3200
LANES = 128
N_CHUNK = M_PAD // LANES
D_PAD = 64
R_BLOCK = 896


def _knn_body(f_ref, mb_ref, out_ref, mba_ref):
    r = f_ref.shape[0]
    d = mb_ref.shape[0]

    @pl.when(pl.program_id(0) == 0)
    def _build_augmented_bank():
        mb = mb_ref[...]
        c_sq = jnp.sum(mb * mb, axis=0, keepdims=True)
        zeros = jnp.zeros((D_PAD - d - 1, M_PAD), jnp.float32)
        mba_ref[...] = jnp.concatenate([-2.0 * mb, c_sq, zeros], axis=0)

    f = f_ref[...]
    f_aug = jnp.concatenate(
        [f, jnp.ones((r, 1), jnp.float32),
         jnp.zeros((r, D_PAD - d - 1), jnp.float32)], axis=1)
    dot = jax.lax.dot_general(
        f_aug, mba_ref[...], (((1,), (0,)), ((), ())),
        preferred_element_type=jnp.float32)

    inf = jnp.float32(jnp.inf)
    lane = jax.lax.broadcasted_iota(jnp.int32, (r, LANES), 1)
    m1 = jnp.full((r, LANES), inf)
    m2 = jnp.full((r, LANES), inf)
    m3 = jnp.full((r, LANES), inf)
    for j in range(N_CHUNK):
        v = dot[:, j * LANES:(j + 1) * LANES]
        if (j + 1) * LANES > M:
            v = jnp.where(lane < M - j * LANES, v, inf)
        t = jnp.minimum(m1, v)
        v = jnp.maximum(m1, v)
        m1 = t
        t = jnp.minimum(m2, v)
        v = jnp.maximum(m2, v)
        m2 = t
        m3 = jnp.minimum(m3, v)

    one = jnp.float32(1.0)
    zero = jnp.float32(0.0)
    a1 = jnp.min(m1, axis=1, keepdims=True)
    e1 = m1 == a1
    n1 = jnp.sum(jnp.where(e1, one, zero), axis=1, keepdims=True)
    u = jnp.where(e1, m2, m1)
    mu = jnp.min(u, axis=1, keepdims=True)
    e2 = u == mu
    n2 = jnp.sum(jnp.where(e2, one, zero), axis=1, keepdims=True)
    v = jnp.where(e2, jnp.where(e1, m3, m2), u)
    mv = jnp.min(v, axis=1, keepdims=True)
    a2 = jnp.where(n1 >= 2, a1, mu)
    a3 = jnp.where(n1 >= 3, a1,
                   jnp.where((n1 == 2) | (n2 >= 2), mu, mv))

    f_sq = jnp.sum(f * f, axis=1, keepdims=True)
    out_ref[:, 0:1] = a1 + f_sq
    out_ref[:, 1:2] = a2 + f_sq
    out_ref[:, 2:3] = a3 + f_sq


@jax.jit
def kernel(target_oriented_features, memory_bank):
    B, HW, D = target_oriented_features.shape
    rows = B * HW
    f = target_oriented_features.reshape(rows, D)
    mb = jnp.pad(memory_bank, ((0, 0), (0, M_PAD - M)))
    out = pl.pallas_call(
        _knn_body,
        grid=(rows // R_BLOCK,),
        in_specs=[
            pl.BlockSpec((R_BLOCK, D), lambda i: (i, 0)),
            pl.BlockSpec((D, M_PAD), lambda i: (0, 0)),
        ],
        out_specs=pl.BlockSpec((R_BLOCK, K_NN), lambda i: (i, 0)),
        out_shape=jax.ShapeDtypeStruct((rows, K_NN), jnp.float32),
        scratch_shapes=[pltpu.VMEM((D_PAD, M_PAD), jnp.float32)],
    )(f, mb)
    return out.reshape(B, HW, K_NN)


# speedup vs baseline: 1.4259x; 1.0029x over previous
import jax
import jax.numpy as jnp
from jax.experimental import pallas as pl
from jax.experimental.pallas import tpu as pltpu

K_NN = 3
M = 3136
M_PAD = 3200
LANES = 128
N_CHUNK = M_PAD // LANES
D_PAD = 64
R_BLOCK = 896
SUB = 64


def _knn_body(f_ref, mb_ref, out_ref, mba_ref):
    r = f_ref.shape[0]
    d = mb_ref.shape[0]

    @pl.when(pl.program_id(0) == 0)
    def _build_augmented_bank():
        mb = mb_ref[...]
        c_sq = jnp.sum(mb * mb, axis=0, keepdims=True)
        zeros = jnp.zeros((D_PAD - d - 1, M_PAD), jnp.float32)
        mba_ref[...] = jnp.concatenate([-2.0 * mb, c_sq, zeros], axis=0)

    f = f_ref[...]
    f_aug = jnp.concatenate(
        [f, jnp.ones((r, 1), jnp.float32),
         jnp.zeros((r, D_PAD - d - 1), jnp.float32)], axis=1)
    dot = jax.lax.dot_general(
        f_aug, mba_ref[...], (((1,), (0,)), ((), ())),
        preferred_element_type=jnp.float32)
    f_sq = jnp.sum(f * f, axis=1, keepdims=True)

    inf = jnp.float32(jnp.inf)
    one = jnp.float32(1.0)
    zero = jnp.float32(0.0)
    lane = jax.lax.broadcasted_iota(jnp.int32, (SUB, LANES), 1)

    for s in range(r // SUB):
        r0 = s * SUB
        m1 = jnp.full((SUB, LANES), inf)
        m2 = jnp.full((SUB, LANES), inf)
        m3 = jnp.full((SUB, LANES), inf)
        for j in range(N_CHUNK):
            v = dot[r0:r0 + SUB, j * LANES:(j + 1) * LANES]
            if (j + 1) * LANES > M:
                v = jnp.where(lane < M - j * LANES, v, inf)
            t = jnp.minimum(m1, v)
            v = jnp.maximum(m1, v)
            m1 = t
            t = jnp.minimum(m2, v)
            v = jnp.maximum(m2, v)
            m2 = t
            m3 = jnp.minimum(m3, v)

        a1 = jnp.min(m1, axis=1, keepdims=True)
        e1 = m1 == a1
        n1 = jnp.sum(jnp.where(e1, one, zero), axis=1, keepdims=True)
        u = jnp.where(e1, m2, m1)
        mu = jnp.min(u, axis=1, keepdims=True)
        e2 = u == mu
        n2 = jnp.sum(jnp.where(e2, one, zero), axis=1, keepdims=True)
        v = jnp.where(e2, jnp.where(e1, m3, m2), u)
        mv = jnp.min(v, axis=1, keepdims=True)
        a2 = jnp.where(n1 >= 2, a1, mu)
        a3 = jnp.where(n1 >= 3, a1,
                       jnp.where((n1 == 2) | (n2 >= 2), mu, mv))

        fsq = f_sq[r0:r0 + SUB, :]
        out_ref[r0:r0 + SUB, 0:1] = a1 + fsq
        out_ref[r0:r0 + SUB, 1:2] = a2 + fsq
        out_ref[r0:r0 + SUB, 2:3] = a3 + fsq


@jax.jit
def kernel(target_oriented_features, memory_bank):
    B, HW, D = target_oriented_features.shape
    rows = B * HW
    f = target_oriented_features.reshape(rows, D)
    mb = jnp.pad(memory_bank, ((0, 0), (0, M_PAD - M)))
    out = pl.pallas_call(
        _knn_body,
        grid=(rows // R_BLOCK,),
        in_specs=[
            pl.BlockSpec((R_BLOCK, D), lambda i: (i, 0)),
            pl.BlockSpec((D, M_PAD), lambda i: (0, 0)),
        ],
        out_specs=pl.BlockSpec((R_BLOCK, K_NN), lambda i: (i, 0)),
        out_shape=jax.ShapeDtypeStruct((rows, K_NN), jnp.float32),
        scratch_shapes=[pltpu.VMEM((D_PAD, M_PAD), jnp.float32)],
    )(f, mb)
    return out.reshape(B, HW, K_NN)


# speedup vs baseline: 1.8613x; 1.3053x over previous
import jax
import jax.numpy as jnp
from jax.experimental import pallas as pl
from jax.experimental.pallas import tpu as pltpu

K_NN = 3
M = 3136
M_PAD = 3200
LANES = 128
N_CHUNK = M_PAD // LANES
D = 56
D_PAD = 64
R_BLOCK = 784
SUB = 56


def _knn_body(f_ref, mb_ref, out_ref, mba_ref):
    @pl.when((pl.program_id(0) == 0) & (pl.program_id(1) == 0))
    def _build_augmented_bank():
        mb = jnp.pad(mb_ref[...], ((0, 0), (0, M_PAD - M)))
        c_sq = jnp.sum(mb * mb, axis=0, keepdims=True)
        zeros = jnp.zeros((D_PAD - D - 1, M_PAD), jnp.float32)
        mba_ref[...] = jnp.concatenate([-2.0 * mb, c_sq, zeros], axis=0)

    f = f_ref[0]
    f_aug = jnp.pad(f, ((0, 0), (0, D_PAD - D)), constant_values=1.0)
    dot = jax.lax.dot_general(
        f_aug, mba_ref[...], (((1,), (0,)), ((), ())),
        preferred_element_type=jnp.float32)
    f_sq = jnp.sum(f * f, axis=1, keepdims=True)

    inf = jnp.float32(jnp.inf)
    one = jnp.float32(1.0)
    zero = jnp.float32(0.0)
    lane = jax.lax.broadcasted_iota(jnp.int32, (SUB, LANES), 1)

    for s in range(R_BLOCK // SUB):
        r0 = s * SUB
        m1 = jnp.full((SUB, LANES), inf)
        m2 = jnp.full((SUB, LANES), inf)
        m3 = jnp.full((SUB, LANES), inf)
        for j in range(N_CHUNK):
            v = dot[r0:r0 + SUB, j * LANES:(j + 1) * LANES]
            if (j + 1) * LANES > M:
                v = jnp.where(lane < M - j * LANES, v, inf)
            t = jnp.minimum(m1, v)
            v = jnp.maximum(m1, v)
            m1 = t
            t = jnp.minimum(m2, v)
            v = jnp.maximum(m2, v)
            m2 = t
            m3 = jnp.minimum(m3, v)

        a1 = jnp.min(m1, axis=1, keepdims=True)
        e1 = m1 == a1
        n1 = jnp.sum(jnp.where(e1, one, zero), axis=1, keepdims=True)
        u = jnp.where(e1, m2, m1)
        mu = jnp.min(u, axis=1, keepdims=True)
        e2 = u == mu
        n2 = jnp.sum(jnp.where(e2, one, zero), axis=1, keepdims=True)
        v = jnp.where(e2, jnp.where(e1, m3, m2), u)
        mv = jnp.min(v, axis=1, keepdims=True)
        a2 = jnp.where(n1 >= 2, a1, mu)
        a3 = jnp.where(n1 >= 3, a1,
                       jnp.where((n1 == 2) | (n2 >= 2), mu, mv))

        fsq = f_sq[r0:r0 + SUB, :]
        out_ref[0, r0:r0 + SUB, 0:1] = a1 + fsq
        out_ref[0, r0:r0 + SUB, 1:2] = a2 + fsq
        out_ref[0, r0:r0 + SUB, 2:3] = a3 + fsq


@jax.jit
def kernel(target_oriented_features, memory_bank):
    B, HW, _ = target_oriented_features.shape
    return pl.pallas_call(
        _knn_body,
        grid=(B, HW // R_BLOCK),
        in_specs=[
            pl.BlockSpec((1, R_BLOCK, D), lambda b, h: (b, h, 0)),
            pl.BlockSpec((D, M), lambda b, h: (0, 0)),
        ],
        out_specs=pl.BlockSpec((1, R_BLOCK, K_NN), lambda b, h: (b, h, 0)),
        out_shape=jax.ShapeDtypeStruct((B, HW, K_NN), jnp.float32),
        scratch_shapes=[pltpu.VMEM((D_PAD, M_PAD), jnp.float32)],
    )(target_oriented_features, memory_bank)


# speedup vs baseline: 1.9442x; 1.0445x over previous
import jax
import jax.numpy as jnp
from jax.experimental import pallas as pl
from jax.experimental.pallas import tpu as pltpu

K_NN = 3
M = 3136
M_PAD = 3200
LANES = 128
N_CHUNK = M_PAD // LANES
D = 56
D_PAD = 64
R_BLOCK = 1568
SUB = 56


def _knn_body(f_ref, mb_ref, out_ref, mba_ref):
    @pl.when((pl.program_id(0) == 0) & (pl.program_id(1) == 0))
    def _build_augmented_bank():
        mb = jnp.pad(mb_ref[...], ((0, 0), (0, M_PAD - M)))
        c_sq = jnp.sum(mb * mb, axis=0, keepdims=True)
        zeros = jnp.zeros((D_PAD - D - 1, M_PAD), jnp.float32)
        mba_ref[...] = jnp.concatenate([-2.0 * mb, c_sq, zeros], axis=0)

    f = f_ref[0]
    f_aug = jnp.pad(f, ((0, 0), (0, D_PAD - D)), constant_values=1.0)
    dot = jax.lax.dot_general(
        f_aug, mba_ref[...], (((1,), (0,)), ((), ())),
        preferred_element_type=jnp.float32)
    f_sq = jnp.sum(f * f, axis=1, keepdims=True)

    inf = jnp.float32(jnp.inf)
    one = jnp.float32(1.0)
    zero = jnp.float32(0.0)
    lane = jax.lax.broadcasted_iota(jnp.int32, (SUB, LANES), 1)

    for s in range(R_BLOCK // SUB):
        r0 = s * SUB
        m1 = jnp.full((SUB, LANES), inf)
        m2 = jnp.full((SUB, LANES), inf)
        m3 = jnp.full((SUB, LANES), inf)
        for j in range(N_CHUNK):
            v = dot[r0:r0 + SUB, j * LANES:(j + 1) * LANES]
            if (j + 1) * LANES > M:
                v = jnp.where(lane < M - j * LANES, v, inf)
            t = jnp.minimum(m1, v)
            v = jnp.maximum(m1, v)
            m1 = t
            t = jnp.minimum(m2, v)
            v = jnp.maximum(m2, v)
            m2 = t
            m3 = jnp.minimum(m3, v)

        a1 = jnp.min(m1, axis=1, keepdims=True)
        e1 = m1 == a1
        n1 = jnp.sum(jnp.where(e1, one, zero), axis=1, keepdims=True)
        u = jnp.where(e1, m2, m1)
        mu = jnp.min(u, axis=1, keepdims=True)
        e2 = u == mu
        n2 = jnp.sum(jnp.where(e2, one, zero), axis=1, keepdims=True)
        v = jnp.where(e2, jnp.where(e1, m3, m2), u)
        mv = jnp.min(v, axis=1, keepdims=True)
        a2 = jnp.where(n1 >= 2, a1, mu)
        a3 = jnp.where(n1 >= 3, a1,
                       jnp.where((n1 == 2) | (n2 >= 2), mu, mv))

        fsq = f_sq[r0:r0 + SUB, :]
        out_ref[0, r0:r0 + SUB, 0:1] = a1 + fsq
        out_ref[0, r0:r0 + SUB, 1:2] = a2 + fsq
        out_ref[0, r0:r0 + SUB, 2:3] = a3 + fsq


@jax.jit
def kernel(target_oriented_features, memory_bank):
    B, HW, _ = target_oriented_features.shape
    return pl.pallas_call(
        _knn_body,
        grid=(B, HW // R_BLOCK),
        in_specs=[
            pl.BlockSpec((1, R_BLOCK, D), lambda b, h: (b, h, 0)),
            pl.BlockSpec((D, M), lambda b, h: (0, 0)),
        ],
        out_specs=pl.BlockSpec((1, R_BLOCK, K_NN), lambda b, h: (b, h, 0)),
        out_shape=jax.ShapeDtypeStruct((B, HW, K_NN), jnp.float32),
        scratch_shapes=[pltpu.VMEM((D_PAD, M_PAD), jnp.float32)],
    )(target_oriented_features, memory_bank)


# speedup vs baseline: 1.9448x; 1.0003x over previous
import jax
import jax.numpy as jnp
from jax.experimental import pallas as pl
from jax.experimental.pallas import tpu as pltpu

K_NN = 3
M = 3136
M_PAD = 3200
LANES = 128
N_CHUNK = M_PAD // LANES
D = 56
D_PAD = 64
R_BLOCK = 1568
SUB = 112


def _knn_body(f_ref, mb_ref, out_ref, mba_ref):
    @pl.when((pl.program_id(0) == 0) & (pl.program_id(1) == 0))
    def _build_augmented_bank():
        mb = jnp.pad(mb_ref[...], ((0, 0), (0, M_PAD - M)))
        c_sq = jnp.sum(mb * mb, axis=0, keepdims=True)
        zeros = jnp.zeros((D_PAD - D - 1, M_PAD), jnp.float32)
        mba_ref[...] = jnp.concatenate([-2.0 * mb, c_sq, zeros], axis=0)

    f = f_ref[0]
    f_aug = jnp.pad(f, ((0, 0), (0, D_PAD - D)), constant_values=1.0)
    dot = jax.lax.dot_general(
        f_aug, mba_ref[...], (((1,), (0,)), ((), ())),
        preferred_element_type=jnp.float32)
    f_sq = jnp.sum(f * f, axis=1, keepdims=True)

    inf = jnp.float32(jnp.inf)
    one = jnp.float32(1.0)
    zero = jnp.float32(0.0)
    lane = jax.lax.broadcasted_iota(jnp.int32, (SUB, LANES), 1)

    for s in range(R_BLOCK // SUB):
        r0 = s * SUB
        m1 = jnp.full((SUB, LANES), inf)
        m2 = jnp.full((SUB, LANES), inf)
        m3 = jnp.full((SUB, LANES), inf)
        for j in range(N_CHUNK):
            v = dot[r0:r0 + SUB, j * LANES:(j + 1) * LANES]
            if (j + 1) * LANES > M:
                v = jnp.where(lane < M - j * LANES, v, inf)
            t = jnp.minimum(m1, v)
            v = jnp.maximum(m1, v)
            m1 = t
            t = jnp.minimum(m2, v)
            v = jnp.maximum(m2, v)
            m2 = t
            m3 = jnp.minimum(m3, v)

        a1 = jnp.min(m1, axis=1, keepdims=True)
        e1 = m1 == a1
        n1 = jnp.sum(jnp.where(e1, one, zero), axis=1, keepdims=True)
        u = jnp.where(e1, m2, m1)
        mu = jnp.min(u, axis=1, keepdims=True)
        e2 = u == mu
        n2 = jnp.sum(jnp.where(e2, one, zero), axis=1, keepdims=True)
        v = jnp.where(e2, jnp.where(e1, m3, m2), u)
        mv = jnp.min(v, axis=1, keepdims=True)
        a2 = jnp.where(n1 >= 2, a1, mu)
        a3 = jnp.where(n1 >= 3, a1,
                       jnp.where((n1 == 2) | (n2 >= 2), mu, mv))

        fsq = f_sq[r0:r0 + SUB, :]
        out_ref[0, r0:r0 + SUB, 0:1] = a1 + fsq
        out_ref[0, r0:r0 + SUB, 1:2] = a2 + fsq
        out_ref[0, r0:r0 + SUB, 2:3] = a3 + fsq


@jax.jit
def kernel(target_oriented_features, memory_bank):
    B, HW, _ = target_oriented_features.shape
    return pl.pallas_call(
        _knn_body,
        grid=(B, HW // R_BLOCK),
        in_specs=[
            pl.BlockSpec((1, R_BLOCK, D), lambda b, h: (b, h, 0)),
            pl.BlockSpec((D, M), lambda b, h: (0, 0)),
        ],
        out_specs=pl.BlockSpec((1, R_BLOCK, K_NN), lambda b, h: (b, h, 0)),
        out_shape=jax.ShapeDtypeStruct((B, HW, K_NN), jnp.float32),
        scratch_shapes=[pltpu.VMEM((D_PAD, M_PAD), jnp.float32)],
    )(target_oriented_features, memory_bank)


# speedup vs baseline: 1.9627x; 1.0092x over previous
import jax
import jax.numpy as jnp
from jax.experimental import pallas as pl
from jax.experimental.pallas import tpu as pltpu

K_NN = 3
M = 3136
M_PAD = 3200
LANES = 128
N_CHUNK = M_PAD // LANES
D = 56
D_PAD = 64
R_BLOCK = 3136
SUB = 112


def _knn_body(f_ref, mb_ref, out_ref, mba_ref):
    @pl.when((pl.program_id(0) == 0) & (pl.program_id(1) == 0))
    def _build_augmented_bank():
        mb = jnp.pad(mb_ref[...], ((0, 0), (0, M_PAD - M)))
        c_sq = jnp.sum(mb * mb, axis=0, keepdims=True)
        zeros = jnp.zeros((D_PAD - D - 1, M_PAD), jnp.float32)
        mba_ref[...] = jnp.concatenate([-2.0 * mb, c_sq, zeros], axis=0)

    f = f_ref[0]
    f_aug = jnp.pad(f, ((0, 0), (0, D_PAD - D)), constant_values=1.0)
    dot = jax.lax.dot_general(
        f_aug, mba_ref[...], (((1,), (0,)), ((), ())),
        preferred_element_type=jnp.float32)
    f_sq = jnp.sum(f * f, axis=1, keepdims=True)

    inf = jnp.float32(jnp.inf)
    one = jnp.float32(1.0)
    zero = jnp.float32(0.0)
    lane = jax.lax.broadcasted_iota(jnp.int32, (SUB, LANES), 1)

    for s in range(R_BLOCK // SUB):
        r0 = s * SUB
        m1 = jnp.full((SUB, LANES), inf)
        m2 = jnp.full((SUB, LANES), inf)
        m3 = jnp.full((SUB, LANES), inf)
        for j in range(N_CHUNK):
            v = dot[r0:r0 + SUB, j * LANES:(j + 1) * LANES]
            if (j + 1) * LANES > M:
                v = jnp.where(lane < M - j * LANES, v, inf)
            t = jnp.minimum(m1, v)
            v = jnp.maximum(m1, v)
            m1 = t
            t = jnp.minimum(m2, v)
            v = jnp.maximum(m2, v)
            m2 = t
            m3 = jnp.minimum(m3, v)

        a1 = jnp.min(m1, axis=1, keepdims=True)
        e1 = m1 == a1
        n1 = jnp.sum(jnp.where(e1, one, zero), axis=1, keepdims=True)
        u = jnp.where(e1, m2, m1)
        mu = jnp.min(u, axis=1, keepdims=True)
        e2 = u == mu
        n2 = jnp.sum(jnp.where(e2, one, zero), axis=1, keepdims=True)
        v = jnp.where(e2, jnp.where(e1, m3, m2), u)
        mv = jnp.min(v, axis=1, keepdims=True)
        a2 = jnp.where(n1 >= 2, a1, mu)
        a3 = jnp.where(n1 >= 3, a1,
                       jnp.where((n1 == 2) | (n2 >= 2), mu, mv))

        fsq = f_sq[r0:r0 + SUB, :]
        out_ref[0, r0:r0 + SUB, 0:1] = a1 + fsq
        out_ref[0, r0:r0 + SUB, 1:2] = a2 + fsq
        out_ref[0, r0:r0 + SUB, 2:3] = a3 + fsq


@jax.jit
def kernel(target_oriented_features, memory_bank):
    B, HW, _ = target_oriented_features.shape
    return pl.pallas_call(
        _knn_body,
        grid=(B, HW // R_BLOCK),
        in_specs=[
            pl.BlockSpec((1, R_BLOCK, D), lambda b, h: (b, h, 0)),
            pl.BlockSpec((D, M), lambda b, h: (0, 0)),
        ],
        out_specs=pl.BlockSpec((1, R_BLOCK, K_NN), lambda b, h: (b, h, 0)),
        out_shape=jax.ShapeDtypeStruct((B, HW, K_NN), jnp.float32),
        scratch_shapes=[pltpu.VMEM((D_PAD, M_PAD), jnp.float32)],
    )(target_oriented_features, memory_bank)
